# baseline (speedup 1.0000x reference)
import functools
import math

import jax
import jax.numpy as jnp
from jax import lax
from jax.experimental import pallas as pl
from jax.experimental.pallas import tpu as pltpu
from jax.experimental.pallas import tpu_sc as plsc

VOCAB = 100000
D = 128
B = 4
T = 2048
NC, NS, L = 2, 16, 16
NW = NC * NS
PW = T // NW
HW = PW // 2
WV = 16
SCALE = math.sqrt(D)

_mesh = plsc.VectorSubcoreMesh(core_axis_name="c", subcore_axis_name="s")


@functools.partial(
    pl.kernel,
    mesh=_mesh,
    out_type=jax.ShapeDtypeStruct((B, T, D), jnp.float32),
    scratch_types=[
        pltpu.VMEM((B, PW), jnp.int32),
        pltpu.VMEM((B * PW, D), jnp.float32),
        pltpu.VMEM((PW, D), jnp.float32),
        pltpu.SemaphoreType.DMA,
        pltpu.SemaphoreType.DMA,
        pltpu.SemaphoreType.DMA,
        pltpu.SemaphoreType.DMA,
        pltpu.SemaphoreType.DMA,
    ],
)
def _embed(idx_hbm, tok_hbm, pos_hbm, out_hbm, idx_v, rows_v, pos_v,
           isem, psem, h0sem, h1sem, osem):
    wid = lax.axis_index("s") * NC + lax.axis_index("c")
    p0 = wid * PW

    icopies = [
        pltpu.async_copy(idx_hbm.at[b, pl.ds(p0, PW)], idx_v.at[b], isem)
        for b in range(B)
    ]
    g0 = []
    for b in range(B):
        icopies[b].wait()
        g0.append(pltpu.async_copy(
            tok_hbm.at[idx_v.at[b, pl.ds(0, HW)]],
            rows_v.at[pl.ds(b * PW, HW)], h0sem))
    pc0 = pltpu.async_copy(
        pos_hbm.at[pl.ds(p0, HW)], pos_v.at[pl.ds(0, HW)], psem)
    g1 = [
        pltpu.async_copy(
            tok_hbm.at[idx_v.at[b, pl.ds(HW, HW)]],
            rows_v.at[pl.ds(b * PW + HW, HW)], h1sem)
        for b in range(B)
    ]
    pc1 = pltpu.async_copy(
        pos_hbm.at[pl.ds(p0 + HW, HW)], pos_v.at[pl.ds(HW, HW)], psem)

    out_waits = []
    for h, (gs, pc) in enumerate(((g0, pc0), (g1, pc1))):
        for g in gs:
            g.wait()
        pc.wait()
        for w in range(HW // WV):
            ibase = h * HW + w * WV

            def body(i, carry, ibase=ibase):
                pi = ibase + i
                for j in range(D // L):
                    sl = pl.ds(j * L, L)
                    pv = pos_v[pi, sl]
                    for b in range(B):
                        row = b * PW + pi
                        rows_v[row, sl] = rows_v[row, sl] * SCALE + pv
                return carry

            lax.fori_loop(0, WV, body, 0)
            for b in range(B):
                out_waits.append(pltpu.async_copy(
                    rows_v.at[pl.ds(b * PW + ibase, WV)],
                    out_hbm.at[b, pl.ds(p0 + ibase, WV)], osem))

    for wt in out_waits:
        wt.wait()


def kernel(token_ids, tok_table, pos_table):
    out = _embed(token_ids.astype(jnp.int32), tok_table, pos_table)
    return out


# speedup vs baseline: 1.0154x; 1.0154x over previous
import functools
import math

import jax
import jax.numpy as jnp
from jax import lax
from jax.experimental import pallas as pl
from jax.experimental.pallas import tpu as pltpu
from jax.experimental.pallas import tpu_sc as plsc

VOCAB = 100000
D = 128
B = 4
T = 2048
NC, NS, L = 2, 16, 16
NW = NC * NS
PW = T // NW
HW = PW // 2
WV = 16
SCALE = math.sqrt(D)

_mesh = plsc.VectorSubcoreMesh(core_axis_name="c", subcore_axis_name="s")


@functools.partial(
    pl.kernel,
    mesh=_mesh,
    out_type=jax.ShapeDtypeStruct((B, T, D), jnp.float32),
    scratch_types=[
        pltpu.VMEM((B, PW), jnp.int32),
        pltpu.VMEM((B * PW, D), jnp.float32),
        pltpu.VMEM((PW, D), jnp.float32),
        pltpu.SemaphoreType.DMA,
        pltpu.SemaphoreType.DMA,
        pltpu.SemaphoreType.DMA,
        pltpu.SemaphoreType.DMA,
        pltpu.SemaphoreType.DMA,
    ],
)
def _embed(idx_hbm, tok_hbm, pos_hbm, out_hbm, idx_v, rows_v, pos_v,
           isem, psem, h0sem, h1sem, osem):
    wid = lax.axis_index("s") * NC + lax.axis_index("c")
    p0 = wid * PW

    pcopy = pltpu.async_copy(pos_hbm.at[pl.ds(p0, PW)], pos_v, psem)
    icopies = [
        pltpu.async_copy(idx_hbm.at[b, pl.ds(p0, PW)], idx_v.at[b], isem)
        for b in range(B)
    ]
    for c in icopies:
        c.wait()
    hsems = (h0sem, h1sem)
    gathers = [
        [
            pltpu.async_copy(
                tok_hbm.at[idx_v.at[b, pl.ds(h * HW, HW)]],
                rows_v.at[pl.ds(b * PW + h * HW, HW)], hsems[h])
            for b in range(B)
        ]
        for h in range(2)
    ]

    out_waits = []
    for h, gs in enumerate(gathers):
        for g in gs:
            g.wait()
        if h == 0:
            pcopy.wait()
        for w in range(HW // WV):
            ibase = h * HW + w * WV

            def body(i, carry, ibase=ibase):
                pi = ibase + i
                for j in range(D // L):
                    sl = pl.ds(j * L, L)
                    pv = pos_v[pi, sl]
                    for b in range(B):
                        row = b * PW + pi
                        rows_v[row, sl] = rows_v[row, sl] * SCALE + pv
                return carry

            lax.fori_loop(0, WV, body, 0)
            for b in range(B):
                out_waits.append(pltpu.async_copy(
                    rows_v.at[pl.ds(b * PW + ibase, WV)],
                    out_hbm.at[b, pl.ds(p0 + ibase, WV)], osem))

    for wt in out_waits:
        wt.wait()


def kernel(token_ids, tok_table, pos_table):
    out = _embed(token_ids.astype(jnp.int32), tok_table, pos_table)
    return out


# speedup vs baseline: 1.0207x; 1.0052x over previous
import functools
import math

import jax
import jax.numpy as jnp
from jax import lax
from jax.experimental import pallas as pl
from jax.experimental.pallas import tpu as pltpu
from jax.experimental.pallas import tpu_sc as plsc

VOCAB = 100000
D = 128
B = 4
T = 2048
NC, NS, L = 2, 16, 16
NW = NC * NS
PW = T // NW
HW = PW // 2
WV = 32
SCALE = math.sqrt(D)

_mesh = plsc.VectorSubcoreMesh(core_axis_name="c", subcore_axis_name="s")


@functools.partial(
    pl.kernel,
    mesh=_mesh,
    out_type=jax.ShapeDtypeStruct((B, T, D), jnp.float32),
    scratch_types=[
        pltpu.VMEM((B, PW), jnp.int32),
        pltpu.VMEM((B * PW, D), jnp.float32),
        pltpu.VMEM((PW, D), jnp.float32),
        pltpu.SemaphoreType.DMA,
        pltpu.SemaphoreType.DMA,
        pltpu.SemaphoreType.DMA,
        pltpu.SemaphoreType.DMA,
        pltpu.SemaphoreType.DMA,
    ],
)
def _embed(idx_hbm, tok_hbm, pos_hbm, out_hbm, idx_v, rows_v, pos_v,
           isem, psem, h0sem, h1sem, osem):
    wid = lax.axis_index("s") * NC + lax.axis_index("c")
    p0 = wid * PW

    pcopy = pltpu.async_copy(pos_hbm.at[pl.ds(p0, PW)], pos_v, psem)
    icopies = [
        pltpu.async_copy(idx_hbm.at[b, pl.ds(p0, PW)], idx_v.at[b], isem)
        for b in range(B)
    ]
    for c in icopies:
        c.wait()
    hsems = (h0sem, h1sem)
    gathers = [
        [
            pltpu.async_copy(
                tok_hbm.at[idx_v.at[b, pl.ds(h * HW, HW)]],
                rows_v.at[pl.ds(b * PW + h * HW, HW)], hsems[h])
            for b in range(B)
        ]
        for h in range(2)
    ]

    out_waits = []
    for h, gs in enumerate(gathers):
        for g in gs:
            g.wait()
        if h == 0:
            pcopy.wait()
        for w in range(HW // WV):
            ibase = h * HW + w * WV

            def body(i, carry, ibase=ibase):
                pi = ibase + i
                for j in range(D // L):
                    sl = pl.ds(j * L, L)
                    pv = pos_v[pi, sl]
                    for b in range(B):
                        row = b * PW + pi
                        rows_v[row, sl] = rows_v[row, sl] * SCALE + pv
                return carry

            lax.fori_loop(0, WV, body, 0)
            for b in range(B):
                out_waits.append(pltpu.async_copy(
                    rows_v.at[pl.ds(b * PW + ibase, WV)],
                    out_hbm.at[b, pl.ds(p0 + ibase, WV)], osem))

    for wt in out_waits:
        wt.wait()


def kernel(token_ids, tok_table, pos_table):
    out = _embed(token_ids.astype(jnp.int32), tok_table, pos_table)
    return out


# speedup vs baseline: 1.0274x; 1.0066x over previous
import functools
import math

import jax
import jax.numpy as jnp
from jax import lax
from jax.experimental import pallas as pl
from jax.experimental.pallas import tpu as pltpu
from jax.experimental.pallas import tpu_sc as plsc

VOCAB = 100000
D = 128
B = 4
T = 2048
NC, NS, L = 2, 16, 16
NW = NC * NS
PW = T // NW
HW = PW // 2
HR = B * HW
SCALE = math.sqrt(D)

_mesh = plsc.VectorSubcoreMesh(core_axis_name="c", subcore_axis_name="s")


@functools.partial(
    pl.kernel,
    mesh=_mesh,
    out_type=jax.ShapeDtypeStruct((B, T, D), jnp.float32),
    scratch_types=[
        pltpu.VMEM((2, HR), jnp.int32),
        pltpu.VMEM((2 * HR, D), jnp.float32),
        pltpu.VMEM((PW, D), jnp.float32),
        pltpu.SemaphoreType.DMA,
        pltpu.SemaphoreType.DMA,
        pltpu.SemaphoreType.DMA,
        pltpu.SemaphoreType.DMA,
        pltpu.SemaphoreType.DMA,
    ],
)
def _embed(idx_hbm, tok_hbm, pos_hbm, out_hbm, idx_v, rows_v, pos_v,
           isem, psem, h0sem, h1sem, osem):
    wid = lax.axis_index("s") * NC + lax.axis_index("c")
    p0 = wid * PW

    pcopy = pltpu.async_copy(pos_hbm.at[pl.ds(p0, PW)], pos_v, psem)
    icopies = [
        pltpu.async_copy(
            idx_hbm.at[b, pl.ds(p0 + h * HW, HW)],
            idx_v.at[h, pl.ds(b * HW, HW)], isem)
        for h in range(2)
        for b in range(B)
    ]
    for c in icopies:
        c.wait()
    hsems = (h0sem, h1sem)
    gathers = [
        pltpu.async_copy(
            tok_hbm.at[idx_v.at[h]],
            rows_v.at[pl.ds(h * HR, HR)], hsems[h])
        for h in range(2)
    ]

    out_waits = []
    for h, g in enumerate(gathers):
        g.wait()
        if h == 0:
            pcopy.wait()

        def body(i, carry, h=h):
            pi = h * HW + i
            for j in range(D // L):
                sl = pl.ds(j * L, L)
                pv = pos_v[pi, sl]
                for b in range(B):
                    row = h * HR + b * HW + i
                    rows_v[row, sl] = rows_v[row, sl] * SCALE + pv
            return carry

        lax.fori_loop(0, HW, body, 0)
        for b in range(B):
            out_waits.append(pltpu.async_copy(
                rows_v.at[pl.ds(h * HR + b * HW, HW)],
                out_hbm.at[b, pl.ds(p0 + h * HW, HW)], osem))

    for wt in out_waits:
        wt.wait()


def kernel(token_ids, tok_table, pos_table):
    out = _embed(token_ids.astype(jnp.int32), tok_table, pos_table)
    return out
